# Initial kernel scaffold; baseline (speedup 1.0000x reference)
import jax
import jax.numpy as jnp
from jax.experimental import pallas as pl


def kernel(node_type, node_row_splits, adjacency, emb_table, W, b):
    raise NotImplementedError("write your pallas kernel here")


# speedup vs baseline: 24.7412x; 24.7412x over previous
import functools

import jax
import jax.numpy as jnp
from jax import lax
from jax.experimental import pallas as pl
from jax.experimental.pallas import tpu as pltpu
from jax.experimental.pallas import tpu_sc as plsc

N_NODES = 10000
N_EDGES = 320000
HID = 128
NUM_TYPES = 128

NC, NS, LANES = 2, 16, 16
NW = NC * NS
EPW = N_EDGES // NW
ROWS = (EPW + 127) // 128
EPW_PAD = ROWS * 128
CWORDS = N_NODES * HID
CWORDS_A = CWORDS + 128
ZCH = 2000
CW_PER_SUB = CWORDS // NS


def _sc_body(nt_hbm, src_hbm, dst_hbm, out_hbm,
             nt_v, src_v, dst_v, idx2, ones_v, zbuf, c_sh, sem):
    c = lax.axis_index("c")
    s = lax.axis_index("s")
    wid = s * NC + c
    base = wid * EPW

    def _zb(i, carry):
        zbuf[pl.ds(i * LANES, LANES)] = jnp.zeros((LANES,), jnp.float32)
        return carry
    lax.fori_loop(0, ZCH // LANES, _zb, 0)
    for k in range(CW_PER_SUB // ZCH):
        pltpu.sync_copy(zbuf, c_sh.at[pl.ds(s * CW_PER_SUB + k * ZCH, ZCH)])

    for l in range(128 // LANES):
        ones_v[pl.ds(l * LANES, LANES)] = jnp.ones((LANES,), jnp.float32)

    pltpu.sync_copy(nt_hbm, nt_v)
    pltpu.sync_copy(src_hbm.at[pl.ds(base, EPW)], src_v.at[pl.ds(0, EPW)])
    pltpu.sync_copy(dst_hbm.at[pl.ds(base, EPW)], dst_v.at[pl.ds(0, EPW)])
    for k in range(EPW // LANES, EPW_PAD // LANES):
        src_v[pl.ds(k * LANES, LANES)] = jnp.zeros((LANES,), jnp.int32)
        dst_v[pl.ds(k * LANES, LANES)] = jnp.zeros((LANES,), jnp.int32)

    def _grp(r, carry):
        for l in range(8):
            off = r * 128 + l * LANES
            sv = src_v[pl.ds(off, LANES)]
            dv = dst_v[pl.ds(off, LANES)]
            tv = plsc.load_gather(nt_v, [sv])
            idx2[r, pl.ds(l * LANES, LANES)] = dv * HID + tv
        return carry
    lax.fori_loop(0, ROWS, _grp, 0)

    dummy16 = jnp.full((LANES,), CWORDS, jnp.int32)
    for l in range((EPW - (ROWS - 1) * 128) // LANES, 8):
        idx2[ROWS - 1, pl.ds(l * LANES, LANES)] = dummy16

    plsc.subcore_barrier()

    def _scat(j, carry):
        pltpu.sync_copy(ones_v, c_sh.at[idx2.at[j]], add=True)
        return carry
    lax.fori_loop(0, ROWS, _scat, 0)

    plsc.subcore_barrier()

    pltpu.sync_copy(c_sh.at[pl.ds(s * CW_PER_SUB, CW_PER_SUB)],
                    out_hbm.at[c, pl.ds(s * CW_PER_SUB, CW_PER_SUB)])


_sc_count = functools.partial(
    pl.kernel,
    mesh=plsc.VectorSubcoreMesh(core_axis_name="c", subcore_axis_name="s"),
    compiler_params=pltpu.CompilerParams(needs_layout_passes=False),
    out_type=jax.ShapeDtypeStruct((NC, CWORDS), jnp.float32),
    scratch_types=[
        pltpu.VMEM((N_NODES,), jnp.int32),
        pltpu.VMEM((EPW_PAD,), jnp.int32),
        pltpu.VMEM((EPW_PAD,), jnp.int32),
        pltpu.VMEM((ROWS, 128), jnp.int32),
        pltpu.VMEM((128,), jnp.float32),
        pltpu.VMEM((ZCH,), jnp.float32),
        pltpu.VMEM_SHARED((CWORDS_A,), jnp.float32),
        pltpu.SemaphoreType.DMA,
    ],
)(_sc_body)


BLK = 1000
NBLK = N_NODES // BLK


def _tc_body(nt_ref, c2_ref, e_ref, w_ref, b_ref, out_ref):
    csum = c2_ref[0] + c2_ref[1]
    deg = jnp.sum(csum, axis=1, keepdims=True)
    agg = jnp.dot(csum, e_ref[...], preferred_element_type=jnp.float32)
    mean = agg / jnp.maximum(deg, 1.0)
    proc = jnp.maximum(
        jnp.dot(mean, w_ref[...], preferred_element_type=jnp.float32)
        + b_ref[...], 0.0)
    nt = nt_ref[0, 0, :]
    ids = lax.broadcasted_iota(jnp.int32, (BLK, NUM_TYPES), 1)
    oh = (nt[:, None] == ids).astype(jnp.float32)
    feat = jnp.dot(oh, e_ref[...], preferred_element_type=jnp.float32)
    out_ref[...] = jnp.concatenate([feat, proc], axis=1)


def _tc_encode(nt3, c2, emb_table, W, b2):
    return pl.pallas_call(
        _tc_body,
        grid=(NBLK,),
        in_specs=[
            pl.BlockSpec((1, 1, BLK), lambda i: (i, 0, 0)),
            pl.BlockSpec((NC, BLK, HID), lambda i: (0, i, 0)),
            pl.BlockSpec((NUM_TYPES, HID), lambda i: (0, 0)),
            pl.BlockSpec((HID, HID), lambda i: (0, 0)),
            pl.BlockSpec((1, HID), lambda i: (0, 0)),
        ],
        out_specs=pl.BlockSpec((BLK, 2 * HID), lambda i: (i, 0)),
        out_shape=jax.ShapeDtypeStruct((N_NODES, 2 * HID), jnp.float32),
    )(nt3, c2, emb_table, W, b2)


def kernel(node_type, node_row_splits, adjacency, emb_table, W, b):
    del node_row_splits
    nt = node_type.astype(jnp.int32)
    adj = adjacency.astype(jnp.int32)
    c2 = _sc_count(nt, adj[0], adj[1])
    c2 = c2.reshape(NC, N_NODES, HID)
    out = _tc_encode(nt.reshape(NBLK, 1, BLK), c2, emb_table, W,
                     b.reshape(1, HID))
    return out


# speedup vs baseline: 31.9006x; 1.2894x over previous
import functools

import jax
import jax.numpy as jnp
from jax import lax
from jax.experimental import pallas as pl
from jax.experimental.pallas import tpu as pltpu
from jax.experimental.pallas import tpu_sc as plsc

N_NODES = 10000
N_EDGES = 320000
HID = 128
NUM_TYPES = 128

NC, NS, LANES = 2, 16, 16
NW = NC * NS
EPW = N_EDGES // NW
ROWS = (EPW + 127) // 128
EPW_PAD = ROWS * 128
CWORDS = N_NODES * HID
CWORDS_A = CWORDS + 128
ZCH = 2000
CW_PER_SUB = CWORDS // NS


def _sc_body(nt_hbm, adj_hbm, out_hbm,
             nt_v, src_v, dst_v, idx2, ones_v, zbuf, c_sh,
             sem_in, sem_z, sem_s):
    c = lax.axis_index("c")
    s = lax.axis_index("s")
    wid = s * NC + c
    base = wid * EPW

    pltpu.make_async_copy(nt_hbm, nt_v, sem_in).start()
    pltpu.make_async_copy(adj_hbm.at[pl.ds(base, EPW)],
                          src_v.at[pl.ds(0, EPW)], sem_in).start()
    pltpu.make_async_copy(adj_hbm.at[pl.ds(N_EDGES + base, EPW)],
                          dst_v.at[pl.ds(0, EPW)], sem_in).start()

    def _zb(i, carry):
        zbuf[pl.ds(i * LANES, LANES)] = jnp.zeros((LANES,), jnp.float32)
        return carry
    lax.fori_loop(0, ZCH // LANES, _zb, 0)
    for l in range(128 // LANES):
        ones_v[pl.ds(l * LANES, LANES)] = jnp.ones((LANES,), jnp.float32)

    def _zfire(k, carry):
        pltpu.make_async_copy(
            zbuf, c_sh.at[pl.ds(s * CW_PER_SUB + k * ZCH, ZCH)],
            sem_z).start()
        return carry
    lax.fori_loop(0, CW_PER_SUB // ZCH, _zfire, 0)

    pltpu.make_async_copy(nt_hbm, nt_v, sem_in).wait()
    pltpu.make_async_copy(adj_hbm.at[pl.ds(base, EPW)],
                          src_v.at[pl.ds(0, EPW)], sem_in).wait()
    pltpu.make_async_copy(adj_hbm.at[pl.ds(N_EDGES + base, EPW)],
                          dst_v.at[pl.ds(0, EPW)], sem_in).wait()
    for k in range(EPW // LANES, EPW_PAD // LANES):
        src_v[pl.ds(k * LANES, LANES)] = jnp.zeros((LANES,), jnp.int32)
        dst_v[pl.ds(k * LANES, LANES)] = jnp.zeros((LANES,), jnp.int32)

    def _grp(r, carry):
        for l in range(8):
            off = r * 128 + l * LANES
            sv = src_v[pl.ds(off, LANES)]
            dv = dst_v[pl.ds(off, LANES)]
            tv = plsc.load_gather(nt_v, [sv])
            idx2[r, pl.ds(l * LANES, LANES)] = dv * HID + tv
        return carry
    lax.fori_loop(0, ROWS, _grp, 0)

    dummy16 = jnp.full((LANES,), CWORDS, jnp.int32)
    for l in range((EPW - (ROWS - 1) * 128) // LANES, 8):
        idx2[ROWS - 1, pl.ds(l * LANES, LANES)] = dummy16

    def _zdrain(k, carry):
        pltpu.make_async_copy(
            zbuf, c_sh.at[pl.ds(s * CW_PER_SUB + k * ZCH, ZCH)],
            sem_z).wait()
        return carry
    lax.fori_loop(0, CW_PER_SUB // ZCH, _zdrain, 0)
    plsc.subcore_barrier()

    def _sfire(j, carry):
        pltpu.make_async_copy(ones_v, c_sh.at[idx2.at[j]], sem_s).start(
            add=True)
        return carry
    lax.fori_loop(0, ROWS, _sfire, 0)

    def _sdrain(j, carry):
        pltpu.make_async_copy(ones_v, c_sh.at[idx2.at[j]], sem_s).wait()
        return carry
    lax.fori_loop(0, ROWS, _sdrain, 0)

    plsc.subcore_barrier()

    pltpu.sync_copy(c_sh.at[pl.ds(s * CW_PER_SUB, CW_PER_SUB)],
                    out_hbm.at[c, pl.ds(s * CW_PER_SUB, CW_PER_SUB)])


_sc_count = functools.partial(
    pl.kernel,
    mesh=plsc.VectorSubcoreMesh(core_axis_name="c", subcore_axis_name="s"),
    compiler_params=pltpu.CompilerParams(needs_layout_passes=False),
    out_type=jax.ShapeDtypeStruct((NC, CWORDS), jnp.float32),
    scratch_types=[
        pltpu.VMEM((N_NODES,), jnp.int32),
        pltpu.VMEM((EPW_PAD,), jnp.int32),
        pltpu.VMEM((EPW_PAD,), jnp.int32),
        pltpu.VMEM((ROWS, 128), jnp.int32),
        pltpu.VMEM((128,), jnp.float32),
        pltpu.VMEM((ZCH,), jnp.float32),
        pltpu.VMEM_SHARED((CWORDS_A,), jnp.float32),
        pltpu.SemaphoreType.DMA,
        pltpu.SemaphoreType.DMA,
        pltpu.SemaphoreType.DMA,
    ],
)(_sc_body)


BLK = 1000
NBLK = N_NODES // BLK


def _tc_body(nt_ref, c2_ref, e_ref, w_ref, b_ref, out_ref):
    csum = c2_ref[0] + c2_ref[1]
    deg = jnp.sum(csum, axis=1, keepdims=True)
    agg = jnp.dot(csum, e_ref[...], preferred_element_type=jnp.float32)
    mean = agg / jnp.maximum(deg, 1.0)
    proc = jnp.maximum(
        jnp.dot(mean, w_ref[...], preferred_element_type=jnp.float32)
        + b_ref[...], 0.0)
    nt = nt_ref[0, 0, :]
    ids = lax.broadcasted_iota(jnp.int32, (BLK, NUM_TYPES), 1)
    oh = (nt[:, None] == ids).astype(jnp.float32)
    feat = jnp.dot(oh, e_ref[...], preferred_element_type=jnp.float32)
    out_ref[...] = jnp.concatenate([feat, proc], axis=1)


def _tc_encode(nt3, c2, emb_table, W, b2):
    return pl.pallas_call(
        _tc_body,
        grid=(NBLK,),
        in_specs=[
            pl.BlockSpec((1, 1, BLK), lambda i: (i, 0, 0)),
            pl.BlockSpec((NC, BLK, HID), lambda i: (0, i, 0)),
            pl.BlockSpec((NUM_TYPES, HID), lambda i: (0, 0)),
            pl.BlockSpec((HID, HID), lambda i: (0, 0)),
            pl.BlockSpec((1, HID), lambda i: (0, 0)),
        ],
        out_specs=pl.BlockSpec((BLK, 2 * HID), lambda i: (i, 0)),
        out_shape=jax.ShapeDtypeStruct((N_NODES, 2 * HID), jnp.float32),
    )(nt3, c2, emb_table, W, b2)


def kernel(node_type, node_row_splits, adjacency, emb_table, W, b):
    del node_row_splits
    nt = node_type.astype(jnp.int32)
    adj = adjacency.astype(jnp.int32)
    c2 = _sc_count(nt, adj.reshape(2 * N_EDGES))
    c2 = c2.reshape(NC, N_NODES, HID)
    out = _tc_encode(nt.reshape(NBLK, 1, BLK), c2, emb_table, W,
                     b.reshape(1, HID))
    return out
